# baseline (speedup 1.0000x reference)
import functools

import jax
import jax.numpy as jnp
from jax import lax
from jax.experimental import pallas as pl
from jax.experimental.pallas import tpu as pltpu
from jax.experimental.pallas import tpu_sc as plsc

NG = 50000
NS = 50000
E = 800000
HID = 64

NSUB = 16
NCH = E // 128
CH_PER_TILE = NCH // NSUB
CH_EXTRA = NCH % NSUB
HALF = NS // 2
ACC_ROWS = 25088
ZROWS_PER_TILE = ACC_ROWS // NSUB
TRASH = 25024
OUT_PER_TILE = 1568
ZCH = 224

_MESH = plsc.VectorSubcoreMesh(core_axis_name="c", subcore_axis_name="s")


def _nchunks(s_idx):
    return jnp.where(s_idx < CH_EXTRA, CH_PER_TILE + 1, CH_PER_TILE)


def _compute_local_dst(dstb, ldb, base):
    def j_body(j, _):
        off = pl.multiple_of(j * 16, 16)
        d16 = dstb[pl.ds(off, 16)]
        ld = d16 - base
        ok = (ld >= 0) & (ld < HALF)
        ldb[pl.ds(off, 16)] = jnp.where(ok, ld, TRASH)
        return 0
    lax.fori_loop(0, 8, j_body, 0)


def _zero_acc(zsrc, acc, s_idx):
    z0 = s_idx * ZROWS_PER_TILE
    for i in range(ZROWS_PER_TILE // ZCH):
        pltpu.sync_copy(zsrc, acc.at[pl.ds(z0 + i * ZCH, ZCH)])


def _write_out(acc, out, c_idx, s_idx):
    @pl.when(s_idx < NSUB - 1)
    def _():
        r0 = pl.multiple_of(s_idx * OUT_PER_TILE, 8)
        o0 = pl.multiple_of(c_idx * HALF + s_idx * OUT_PER_TILE, 8)
        pltpu.sync_copy(acc.at[pl.ds(r0, OUT_PER_TILE)],
                        out.at[pl.ds(o0, OUT_PER_TILE)])

    last0 = (NSUB - 1) * OUT_PER_TILE
    @pl.when(s_idx == NSUB - 1)
    def _():
        o0 = pl.multiple_of(c_idx * HALF + last0, 8)
        pltpu.sync_copy(acc.at[pl.ds(last0, HALF - last0)],
                        out.at[pl.ds(o0, HALF - last0)])


def _scale_rows(rows, wref, w, densb=None):
    ngrp = w // 16

    def g_body(jg, _):
        off = pl.multiple_of(jg * 16, 16)
        w16 = wref[pl.ds(off, 16)]
        for r in range(16):
            sp = jnp.full((16,), w16[r], jnp.float32)
            row = jg * 16 + r
            if densb is not None:
                densb[row, :] = sp
            for j in range(ngrp):
                o2 = pl.multiple_of(j * 16, 16)
                rows[row, pl.ds(o2, 16)] = rows[row, pl.ds(o2, 16)] * sp
        return 0
    lax.fori_loop(0, 8, g_body, 0)


def _make_seg_kernel(mode, w):
    with_rows = mode in ("plain", "norm", "gat")
    with_cnt = mode in ("plain", "count")
    cw = 8 if with_cnt else 16

    out_type = []
    if with_rows:
        out_type.append(jax.ShapeDtypeStruct((NS, w), jnp.float32))
    if with_cnt or mode == "wcount":
        out_type.append(jax.ShapeDtypeStruct((NS, cw), jnp.float32))

    scratch = [pltpu.VMEM((128,), jnp.int32)]
    scratch.append(pltpu.VMEM((128,), jnp.int32))
    if with_rows or mode == "wcount":
        scratch.append(pltpu.VMEM((128,), jnp.int32))
    if with_rows:
        scratch.append(pltpu.VMEM((128, w), jnp.float32))
        scratch.append(pltpu.VMEM_SHARED((ACC_ROWS, w), jnp.float32))
    if with_cnt:
        scratch.append(pltpu.VMEM_SHARED((ACC_ROWS, 8), jnp.float32))
        scratch.append(pltpu.VMEM((128, 8), jnp.float32))
    if mode == "wcount":
        scratch.append(pltpu.VMEM_SHARED((ACC_ROWS, 16), jnp.float32))
        scratch.append(pltpu.VMEM((128, 16), jnp.float32))
    if mode == "norm":
        scratch.append(pltpu.VMEM((128,), jnp.float32))
        scratch.append(pltpu.VMEM((128,), jnp.float32))
        scratch.append(pltpu.VMEM((128,), jnp.float32))
    if mode in ("gat", "wcount"):
        scratch.append(pltpu.VMEM((128,), jnp.float32))
        scratch.append(pltpu.VMEM((128,), jnp.float32))
        scratch.append(pltpu.VMEM((128,), jnp.float32))
        scratch.append(pltpu.VMEM((128,), jnp.float32))
        scratch.append(pltpu.VMEM((16,), jnp.float32))
    scratch.append(pltpu.SemaphoreType.DMA)
    scratch.append(pltpu.SemaphoreType.DMA)
    scratch.append(pltpu.SemaphoreType.DMA)

    def body(*refs):
        it = iter(refs)
        if with_rows:
            table = next(it)
        if mode == "norm":
            wnode = next(it)
        if mode in ("gat", "wcount"):
            asn, adn, aef, bnd = next(it), next(it), next(it), next(it)
        srcf = next(it) if (with_rows or mode == "wcount") else None
        dstf = next(it)
        zrows = next(it) if with_rows else None
        zcnt = next(it) if (with_cnt or mode == "wcount") else None
        ones8 = next(it) if with_cnt else None
        out_rows = next(it) if with_rows else None
        out_cnt = next(it) if (with_cnt or mode == "wcount") else None
        dstb = next(it)
        ldb = next(it)
        srcb = next(it) if (with_rows or mode == "wcount") else None
        if with_rows:
            rows = next(it)
            acc = next(it)
        if with_cnt:
            cacc = next(it)
            onesb = next(it)
        if mode == "wcount":
            cacc = next(it)
            densb = next(it)
        if mode == "norm":
            wsv, wdv, wb = next(it), next(it), next(it)
        if mode in ("gat", "wcount"):
            asv, adv, aev, exb, bndv = (next(it), next(it), next(it),
                                        next(it), next(it))
        sem = next(it)
        sem2 = next(it)
        sem3 = next(it)

        c_idx = lax.axis_index("c")
        s_idx = lax.axis_index("s")
        base = c_idx * HALF

        if with_rows:
            _zero_acc(zrows, acc, s_idx)
        if with_cnt or mode == "wcount":
            _zero_acc(zcnt, cacc, s_idx)
        if with_cnt:
            pltpu.sync_copy(ones8, onesb)
        if mode in ("gat", "wcount"):
            pltpu.sync_copy(bnd, bndv)
        plsc.subcore_barrier()

        def alpha_chunk():
            bb = bndv[...]
            def jg_body(j, _):
                off = pl.multiple_of(j * 16, 16)
                t = asv[pl.ds(off, 16)] + adv[pl.ds(off, 16)] + aev[pl.ds(off, 16)]
                t = jnp.maximum(t, 0.2 * t)
                exb[pl.ds(off, 16)] = jnp.exp(t - bb)
                return 0
            lax.fori_loop(0, 8, jg_body, 0)

        def chunk_body(k, _):
            chunk = s_idx + NSUB * k
            e0 = pl.multiple_of(chunk * 128, 128)
            pltpu.sync_copy(dstf.at[pl.ds(e0, 128)], dstb)
            if srcb is not None:
                pltpu.sync_copy(srcf.at[pl.ds(e0, 128)], srcb)
            if mode in ("gat", "wcount"):
                pltpu.sync_copy(aef.at[pl.ds(e0, 128)], aev)
            _compute_local_dst(dstb, ldb, base)

            if with_rows:
                h = pltpu.async_copy(table.at[srcb], rows, sem)
            if mode == "norm":
                h2 = pltpu.async_copy(wnode.at[srcb], wsv, sem2)
                h3 = pltpu.async_copy(wnode.at[dstb], wdv, sem3)
                h2.wait(); h3.wait()
            if mode in ("gat", "wcount"):
                h2 = pltpu.async_copy(asn.at[srcb], asv, sem2)
                h3 = pltpu.async_copy(adn.at[dstb], adv, sem3)
                h2.wait(); h3.wait()
            if with_rows:
                h.wait()

            if mode == "norm":
                def jn_body(j, _):
                    off = pl.multiple_of(j * 16, 16)
                    wb[pl.ds(off, 16)] = wsv[pl.ds(off, 16)] * wdv[pl.ds(off, 16)]
                    return 0
                lax.fori_loop(0, 8, jn_body, 0)
                _scale_rows(rows, wb, w)
            if mode == "gat":
                alpha_chunk()
                _scale_rows(rows, exb, w)
            if mode == "wcount":
                alpha_chunk()
                def js_body(jg, _):
                    off = pl.multiple_of(jg * 16, 16)
                    w16 = exb[pl.ds(off, 16)]
                    for r in range(16):
                        densb[jg * 16 + r, :] = jnp.full((16,), w16[r], jnp.float32)
                    return 0
                lax.fori_loop(0, 8, js_body, 0)

            if with_rows:
                pltpu.sync_copy(rows, acc.at[ldb], add=True)
            if with_cnt:
                pltpu.sync_copy(onesb, cacc.at[ldb], add=True)
            if mode == "wcount":
                pltpu.sync_copy(densb, cacc.at[ldb], add=True)
            return 0

        lax.fori_loop(0, _nchunks(s_idx), chunk_body, 0)
        plsc.subcore_barrier()
        if with_rows:
            _write_out(acc, out_rows, c_idx, s_idx)
        if with_cnt or mode == "wcount":
            _write_out(cacc, out_cnt, c_idx, s_idx)

    if len(out_type) == 1:
        out_type = out_type[0]
    return functools.partial(
        pl.kernel, mesh=_MESH, out_type=out_type, scratch_types=scratch,
        compiler_params=pltpu.CompilerParams(use_tc_tiling_on_sc=False),
    )(body)


_seg_plain_64 = _make_seg_kernel("plain", 64)
_seg_plain_16 = _make_seg_kernel("plain", 16)
_seg_norm_16 = _make_seg_kernel("norm", 16)
_seg_gat = _make_seg_kernel("gat", 64)
_seg_count = _make_seg_kernel("count", 0)
_seg_wcount = _make_seg_kernel("wcount", 0)

_SM_ROWS = 392


def _softmax_body(x_ref, o_ref):
    x = x_ref[...]
    ridx = lax.broadcasted_iota(jnp.int32, x.shape, 0)
    cidx = lax.broadcasted_iota(jnp.int32, x.shape, 1)
    valid = (ridx * 128 + cidx) < NS
    xm = jnp.where(valid, x, -jnp.inf)
    m = jnp.max(xm)
    e = jnp.where(valid, jnp.exp(x - m), 0.0)
    o_ref[...] = e / jnp.sum(e)


def _softmax_over_all(logit):
    x = jnp.zeros((_SM_ROWS * 128,), jnp.float32).at[:NS].set(logit)
    x = x.reshape(_SM_ROWS, 128)
    y = pl.pallas_call(
        _softmax_body,
        out_shape=jax.ShapeDtypeStruct((_SM_ROWS, 128), jnp.float32),
    )(x)
    return y.reshape(-1)[:NS]


def _pad16(x):
    return jnp.pad(x, ((0, 0), (0, 16 - x.shape[1])))


_BLK = 1000
_NBLK = NS // _BLK


def _rs(w):
    return pl.BlockSpec((_BLK, w), lambda i: (i, 0))


def _fs(shape):
    return pl.BlockSpec(shape, lambda i: (0,) * len(shape))


def _accmax(i, ref, val):
    @pl.when(i == 0)
    def _():
        ref[...] = jnp.full((1, 1), val, jnp.float32)

    @pl.when(i > 0)
    def _():
        ref[...] = jnp.maximum(ref[...], val)


def _dis_body(deg_ref, o_ref):
    col = deg_ref[...][:, 0:1]
    o_ref[...] = jnp.where(col > 0, lax.rsqrt(jnp.maximum(col, 1.0)), 0.0)


def _dis_kernel(deg8):
    return pl.pallas_call(
        _dis_body,
        grid=(_NBLK,),
        in_specs=[_rs(8)],
        out_specs=_rs(1),
        out_shape=jax.ShapeDtypeStruct((NG, 1), jnp.float32),
    )(deg8)[:, 0]


def _game_body(x_ref, h1_ref, h2_ref, w0_ref, w1_ref, w2_ref, b_ref, ws_ref,
               as_ref, g_ref, hs_ref, asn_ref, mx_ref):
    i = pl.program_id(0)
    out = (jnp.dot(x_ref[...], w0_ref[...])
           + jnp.dot(h1_ref[...][:, :5], w1_ref[...])
           + jnp.dot(h2_ref[...][:, :5], w2_ref[...]) + b_ref[...])
    g = jnp.maximum(out, 0.0)
    g_ref[...] = g
    hs = jnp.dot(g, ws_ref[...])
    hs_ref[...] = hs
    asn = jnp.dot(hs, as_ref[...])
    asn_ref[...] = asn
    _accmax(i, mx_ref, jnp.max(asn))


def _game_kernel(x_game, h1, h2, w0, w1, w2, b, ws, a_s):
    return pl.pallas_call(
        _game_body,
        grid=(_NBLK,),
        in_specs=[_rs(5), _rs(16), _rs(16), _fs((5, 64)), _fs((5, 64)),
                  _fs((5, 64)), _fs((1, 64)), _fs((64, 64)), _fs((64, 1))],
        out_specs=[_rs(64), _rs(64), _rs(1), pl.BlockSpec((1, 1), lambda i: (0, 0))],
        out_shape=[jax.ShapeDtypeStruct((NG, 64), jnp.float32),
                   jax.ShapeDtypeStruct((NG, 64), jnp.float32),
                   jax.ShapeDtypeStruct((NG, 1), jnp.float32),
                   jax.ShapeDtypeStruct((1, 1), jnp.float32)],
    )(x_game, h1, h2, w0, w1, w2, b[None, :], ws, a_s[:, None])


def _state_body(ssum_ref, cnt_ref, x_ref, wl_ref, wr_ref, b_ref, wdad_ref,
                s2_ref, adn_ref, mx_ref):
    i = pl.program_id(0)
    cnt = jnp.maximum(cnt_ref[...][:, 0:1], 1.0)
    mean = ssum_ref[...][:, :6] / cnt
    s2 = jnp.maximum(jnp.dot(mean, wl_ref[...])
                     + jnp.dot(x_ref[...], wr_ref[...]) + b_ref[...], 0.0)
    s2_ref[...] = s2
    adn = jnp.dot(s2, wdad_ref[...])
    adn_ref[...] = adn
    _accmax(i, mx_ref, jnp.max(adn))


def _state_kernel(s_sum, cnt8, x_state, wl, wr, b, wd_ad):
    return pl.pallas_call(
        _state_body,
        grid=(_NBLK,),
        in_specs=[_rs(16), _rs(8), _rs(6), _fs((6, 64)), _fs((6, 64)),
                  _fs((1, 64)), _fs((64, 1))],
        out_specs=[_rs(64), _rs(1), pl.BlockSpec((1, 1), lambda i: (0, 0))],
        out_shape=[jax.ShapeDtypeStruct((NS, 64), jnp.float32),
                   jax.ShapeDtypeStruct((NS, 1), jnp.float32),
                   jax.ShapeDtypeStruct((1, 1), jnp.float32)],
    )(s_sum, cnt8, x_state, wl, wr, b[None, :], wd_ad[:, None])


_EBLK = 8000


def _edge_body(ea_ref, c_ref, aen_ref, mx_ref):
    i = pl.program_id(0)
    ea = ea_ref[...]
    c = c_ref[...]
    v = ea[:, 0:1] * c[0, 0] + ea[:, 1:2] * c[0, 1]
    aen_ref[...] = v
    _accmax(i, mx_ref, jnp.max(v))


def _edge_kernel(edge_attr, we_ae):
    return pl.pallas_call(
        _edge_body,
        grid=(E // _EBLK,),
        in_specs=[pl.BlockSpec((_EBLK, 2), lambda i: (i, 0)), _fs((1, 2))],
        out_specs=[pl.BlockSpec((_EBLK, 1), lambda i: (i, 0)),
                   pl.BlockSpec((1, 1), lambda i: (0, 0))],
        out_shape=[jax.ShapeDtypeStruct((E, 1), jnp.float32),
                   jax.ShapeDtypeStruct((1, 1), jnp.float32)],
    )(edge_attr, we_ae[None, :])


def _final_body(u_ref, den_ref, isum_ref, icnt_ref, gb_ref, wl_ref, wr_ref,
                b4_ref, lw_ref, lb_ref, llw_ref, llb_ref, logit_ref):
    den = jnp.maximum(den_ref[...][:, 0:1], 1e-16)
    s3 = jnp.maximum(u_ref[...] / den + gb_ref[...], 0.0)
    icnt = jnp.maximum(icnt_ref[...][:, 0:1], 1.0)
    mean = isum_ref[...] / icnt
    s4 = jnp.maximum(jnp.dot(mean, wl_ref[...])
                     + jnp.dot(s3, wr_ref[...]) + b4_ref[...], 0.0)
    s5 = jnp.dot(s4, lw_ref[...]) + lb_ref[...]
    logit_ref[...] = jnp.dot(s5, llw_ref[...]) + llb_ref[...]


def _final_kernel(u, den16, in_sum, icnt8, gb, wl, wr, b4, lw, lb, llw, llb):
    return pl.pallas_call(
        _final_body,
        grid=(_NBLK,),
        in_specs=[_rs(64), _rs(16), _rs(64), _rs(8), _fs((1, 64)),
                  _fs((64, 64)), _fs((64, 64)), _fs((1, 64)), _fs((64, 8)),
                  _fs((1, 8)), _fs((8, 1)), _fs((1, 1))],
        out_specs=_rs(1),
        out_shape=jax.ShapeDtypeStruct((NS, 1), jnp.float32),
    )(u, den16, in_sum, icnt8, gb[None, :], wl, wr, b4[None, :], lw,
      lb[None, :], llw, llb[None, :])


def kernel(x_game, x_state, edge_attr, params, edge_index_gg, edge_index_ss, edge_index_hist, edge_index_in):
    p = params
    e2 = lambda v: v
    _Z64 = jnp.zeros((ZCH, 64), jnp.float32)
    _Z16 = jnp.zeros((ZCH, 16), jnp.float32)
    _Z8 = jnp.zeros((ZCH, 8), jnp.float32)
    _ONES8 = jnp.ones((128, 8), jnp.float32)
    fold = lambda c: c[:, 0]

    deg8 = _seg_count(e2(edge_index_gg[1]), _Z8, _ONES8)
    dis = _dis_kernel(deg8)
    x16 = _pad16(x_game)
    h1 = _seg_norm_16(x16, dis, e2(edge_index_gg[0]), e2(edge_index_gg[1]), _Z16)
    h2 = _seg_norm_16(h1, dis, e2(edge_index_gg[0]), e2(edge_index_gg[1]), _Z16)

    s_sum, cnt8 = _seg_plain_16(_pad16(x_state), e2(edge_index_ss[0]),
                                e2(edge_index_ss[1]), _Z16, _Z8, _ONES8)

    g, hs, asn1, asmax = _game_kernel(x_game, h1, h2, p['tag_W'][0],
                                      p['tag_W'][1], p['tag_W'][2],
                                      p['tag_b'], p['g_Ws'], p['g_as'])
    s2, adn1, admax = _state_kernel(s_sum, cnt8, x_state, p['s2_Wl'],
                                    p['s2_Wr'], p['s2_b'],
                                    p['g_Wd'] @ p['g_ad'])
    aen1, aemax = _edge_kernel(edge_attr, p['g_We'] @ p['g_ae'])

    bnd = asmax[0, 0] + admax[0, 0] + aemax[0, 0]
    bnd = jnp.maximum(bnd, 0.2 * bnd)
    bnd16 = jnp.full((16,), bnd)
    asn, adn, aen = asn1[:, 0], adn1[:, 0], aen1[:, 0]
    u = _seg_gat(hs, asn, adn, e2(aen), bnd16,
                 e2(edge_index_hist[0]), e2(edge_index_hist[1]), _Z64)
    den = _seg_wcount(asn, adn, e2(aen), bnd16,
                      e2(edge_index_hist[0]), e2(edge_index_hist[1]), _Z16)

    in_sum, icnt8 = _seg_plain_64(g, e2(edge_index_in[0]),
                                  e2(edge_index_in[1]), _Z64, _Z8, _ONES8)

    logit = _final_kernel(u, den, in_sum, icnt8, p['g_b'], p['s4_Wl'],
                          p['s4_Wr'], p['s4_b'], p['lin_W'], p['lin_b'],
                          p['ll_W'], p['ll_b'])
    state_out = _softmax_over_all(logit[:, 0])[:, None]
    return (state_out, x_game)


# speedup vs baseline: 1.2687x; 1.2687x over previous
import functools

import jax
import jax.numpy as jnp
from jax import lax
from jax.experimental import pallas as pl
from jax.experimental.pallas import tpu as pltpu
from jax.experimental.pallas import tpu_sc as plsc

NG = 50000
NS = 50000
E = 800000
HID = 64

NSUB = 16
NCH = E // 128
CH_PER_TILE = NCH // NSUB
CH_EXTRA = NCH % NSUB
HALF = NS // 2
ACC_ROWS = 25088
ZROWS_PER_TILE = ACC_ROWS // NSUB
TRASH = 25024
OUT_PER_TILE = 1568
ZCH = 224

_MESH = plsc.VectorSubcoreMesh(core_axis_name="c", subcore_axis_name="s")

ACC2_ROWS = 50176
Z2_PER_TILE = ACC2_ROWS // NSUB
OUT2_PER_TILE = 3136
NCH2 = NCH // 2
CH2_PER_TILE = NCH2 // NSUB
CH2_EXTRA = NCH2 % NSUB


def _nchunks(s_idx):
    return jnp.where(s_idx < CH_EXTRA, CH_PER_TILE + 1, CH_PER_TILE)


def _nchunks2(s_idx):
    return jnp.where(s_idx < CH2_EXTRA, CH2_PER_TILE + 1, CH2_PER_TILE)


def _compute_local_dst(dstb, ldb, base):
    def j_body(j, _):
        off = pl.multiple_of(j * 16, 16)
        d16 = dstb[pl.ds(off, 16)]
        ld = d16 - base
        ok = (ld >= 0) & (ld < HALF)
        ldb[pl.ds(off, 16)] = jnp.where(ok, ld, TRASH)
        return 0
    lax.fori_loop(0, 8, j_body, 0)


def _zero_acc(zsrc, acc, s_idx, per_tile=ZROWS_PER_TILE):
    z0 = s_idx * per_tile
    for i in range(per_tile // ZCH):
        pltpu.sync_copy(zsrc, acc.at[pl.ds(z0 + i * ZCH, ZCH)])


def _write_out(acc, out, c_idx, s_idx, fr=False):
    per = OUT2_PER_TILE if fr else OUT_PER_TILE
    tot = NS if fr else HALF

    @pl.when(s_idx < NSUB - 1)
    def _():
        r0 = pl.multiple_of(s_idx * per, 8)
        if fr:
            pltpu.sync_copy(acc.at[pl.ds(r0, per)],
                            out.at[c_idx, pl.ds(r0, per)])
        else:
            o0 = pl.multiple_of(c_idx * HALF + r0, 8)
            pltpu.sync_copy(acc.at[pl.ds(r0, per)], out.at[pl.ds(o0, per)])

    last0 = (NSUB - 1) * per
    @pl.when(s_idx == NSUB - 1)
    def _():
        if fr:
            pltpu.sync_copy(acc.at[pl.ds(last0, tot - last0)],
                            out.at[c_idx, pl.ds(last0, tot - last0)])
        else:
            o0 = pl.multiple_of(c_idx * HALF + last0, 8)
            pltpu.sync_copy(acc.at[pl.ds(last0, tot - last0)],
                            out.at[pl.ds(o0, tot - last0)])


def _scale_rows(rows, wref, w, densb=None):
    ngrp = w // 16

    def g_body(jg, _):
        off = pl.multiple_of(jg * 16, 16)
        w16 = wref[pl.ds(off, 16)]
        for r in range(16):
            sp = jnp.full((16,), w16[r], jnp.float32)
            row = jg * 16 + r
            if densb is not None:
                densb[row, :] = sp
            for j in range(ngrp):
                o2 = pl.multiple_of(j * 16, 16)
                rows[row, pl.ds(o2, 16)] = rows[row, pl.ds(o2, 16)] * sp
        return 0
    lax.fori_loop(0, 8, g_body, 0)


def _make_seg_kernel(mode, w, fr=False):
    with_rows = mode in ("plain", "norm", "gat")
    with_cnt = mode in ("plain", "count")
    cw = 8 if with_cnt else 16
    arows = ACC2_ROWS if fr else ACC_ROWS
    oshape = (lambda ww: (2, NS, ww)) if fr else (lambda ww: (NS, ww))

    out_type = []
    if with_rows:
        out_type.append(jax.ShapeDtypeStruct(oshape(w), jnp.float32))
    if with_cnt or mode == "wcount":
        out_type.append(jax.ShapeDtypeStruct(oshape(cw), jnp.float32))

    scratch = [pltpu.VMEM((128,), jnp.int32)]
    scratch.append(pltpu.VMEM((128,), jnp.int32))
    if with_rows or mode == "wcount":
        scratch.append(pltpu.VMEM((128,), jnp.int32))
    if with_rows:
        scratch.append(pltpu.VMEM((128, w), jnp.float32))
        scratch.append(pltpu.VMEM_SHARED((arows, w), jnp.float32))
    if with_cnt:
        scratch.append(pltpu.VMEM_SHARED((arows, 8), jnp.float32))
        scratch.append(pltpu.VMEM((128, 8), jnp.float32))
    if mode == "wcount":
        scratch.append(pltpu.VMEM_SHARED((arows, 16), jnp.float32))
        scratch.append(pltpu.VMEM((128, 16), jnp.float32))
    if mode == "norm":
        scratch.append(pltpu.VMEM((128,), jnp.float32))
        scratch.append(pltpu.VMEM((128,), jnp.float32))
        scratch.append(pltpu.VMEM((128,), jnp.float32))
    if mode in ("gat", "wcount"):
        scratch.append(pltpu.VMEM((128,), jnp.float32))
        scratch.append(pltpu.VMEM((128,), jnp.float32))
        scratch.append(pltpu.VMEM((128,), jnp.float32))
        scratch.append(pltpu.VMEM((128,), jnp.float32))
        scratch.append(pltpu.VMEM((16,), jnp.float32))
    scratch.append(pltpu.SemaphoreType.DMA)
    scratch.append(pltpu.SemaphoreType.DMA)
    scratch.append(pltpu.SemaphoreType.DMA)

    def body(*refs):
        it = iter(refs)
        if with_rows:
            table = next(it)
        if mode == "norm":
            wnode = next(it)
        if mode in ("gat", "wcount"):
            asn, adn, aef, bnd = next(it), next(it), next(it), next(it)
        srcf = next(it) if (with_rows or mode == "wcount") else None
        dstf = next(it)
        zrows = next(it) if with_rows else None
        zcnt = next(it) if (with_cnt or mode == "wcount") else None
        ones8 = next(it) if with_cnt else None
        out_rows = next(it) if with_rows else None
        out_cnt = next(it) if (with_cnt or mode == "wcount") else None
        dstb = next(it)
        ldb = next(it)
        srcb = next(it) if (with_rows or mode == "wcount") else None
        if with_rows:
            rows = next(it)
            acc = next(it)
        if with_cnt:
            cacc = next(it)
            onesb = next(it)
        if mode == "wcount":
            cacc = next(it)
            densb = next(it)
        if mode == "norm":
            wsv, wdv, wb = next(it), next(it), next(it)
        if mode in ("gat", "wcount"):
            asv, adv, aev, exb, bndv = (next(it), next(it), next(it),
                                        next(it), next(it))
        sem = next(it)
        sem2 = next(it)
        sem3 = next(it)

        c_idx = lax.axis_index("c")
        s_idx = lax.axis_index("s")
        base = c_idx * HALF
        zper = Z2_PER_TILE if fr else ZROWS_PER_TILE

        if with_rows:
            _zero_acc(zrows, acc, s_idx, zper)
        if with_cnt or mode == "wcount":
            _zero_acc(zcnt, cacc, s_idx, zper)
        if with_cnt:
            pltpu.sync_copy(ones8, onesb)
        if mode in ("gat", "wcount"):
            pltpu.sync_copy(bnd, bndv)
        plsc.subcore_barrier()

        def alpha_chunk():
            bb = bndv[...]
            def jg_body(j, _):
                off = pl.multiple_of(j * 16, 16)
                t = asv[pl.ds(off, 16)] + adv[pl.ds(off, 16)] + aev[pl.ds(off, 16)]
                t = jnp.maximum(t, 0.2 * t)
                exb[pl.ds(off, 16)] = jnp.exp(t - bb)
                return 0
            lax.fori_loop(0, 8, jg_body, 0)

        def chunk_body(k, _):
            if fr:
                chunk = c_idx * NCH2 + s_idx + NSUB * k
            else:
                chunk = s_idx + NSUB * k
            e0 = pl.multiple_of(chunk * 128, 128)
            pltpu.sync_copy(dstf.at[pl.ds(e0, 128)], dstb)
            if srcb is not None:
                pltpu.sync_copy(srcf.at[pl.ds(e0, 128)], srcb)
            if mode in ("gat", "wcount"):
                pltpu.sync_copy(aef.at[pl.ds(e0, 128)], aev)
            sidx = dstb if fr else ldb
            if not fr:
                _compute_local_dst(dstb, ldb, base)

            if with_rows:
                h = pltpu.async_copy(table.at[srcb], rows, sem)
            if mode == "norm":
                h2 = pltpu.async_copy(wnode.at[srcb], wsv, sem2)
                h3 = pltpu.async_copy(wnode.at[dstb], wdv, sem3)
                h2.wait(); h3.wait()
            if mode in ("gat", "wcount"):
                h2 = pltpu.async_copy(asn.at[srcb], asv, sem2)
                h3 = pltpu.async_copy(adn.at[dstb], adv, sem3)
                h2.wait(); h3.wait()
            if with_rows:
                h.wait()

            if mode == "norm":
                def jn_body(j, _):
                    off = pl.multiple_of(j * 16, 16)
                    wb[pl.ds(off, 16)] = wsv[pl.ds(off, 16)] * wdv[pl.ds(off, 16)]
                    return 0
                lax.fori_loop(0, 8, jn_body, 0)
                _scale_rows(rows, wb, w)
            if mode == "gat":
                alpha_chunk()
                _scale_rows(rows, exb, w)
            if mode == "wcount":
                alpha_chunk()
                def js_body(jg, _):
                    off = pl.multiple_of(jg * 16, 16)
                    w16 = exb[pl.ds(off, 16)]
                    for r in range(16):
                        densb[jg * 16 + r, :] = jnp.full((16,), w16[r], jnp.float32)
                    return 0
                lax.fori_loop(0, 8, js_body, 0)

            if with_rows:
                pltpu.sync_copy(rows, acc.at[sidx], add=True)
            if with_cnt:
                pltpu.sync_copy(onesb, cacc.at[sidx], add=True)
            if mode == "wcount":
                pltpu.sync_copy(densb, cacc.at[sidx], add=True)
            return 0

        nk = _nchunks2(s_idx) if fr else _nchunks(s_idx)
        lax.fori_loop(0, nk, chunk_body, 0)
        plsc.subcore_barrier()
        if with_rows:
            _write_out(acc, out_rows, c_idx, s_idx, fr)
        if with_cnt or mode == "wcount":
            _write_out(cacc, out_cnt, c_idx, s_idx, fr)

    if len(out_type) == 1:
        out_type = out_type[0]
    return functools.partial(
        pl.kernel, mesh=_MESH, out_type=out_type, scratch_types=scratch,
        compiler_params=pltpu.CompilerParams(use_tc_tiling_on_sc=False),
    )(body)


_seg_plain_64 = _make_seg_kernel("plain", 64)
_seg_plain_16 = _make_seg_kernel("plain", 16, fr=True)
_seg_norm_16 = _make_seg_kernel("norm", 16, fr=True)
_seg_gat = _make_seg_kernel("gat", 64)
_seg_count = _make_seg_kernel("count", 0, fr=True)
_seg_wcount = _make_seg_kernel("wcount", 0, fr=True)

_SM_ROWS = 392


def _softmax_body(x_ref, o_ref):
    x = x_ref[...]
    ridx = lax.broadcasted_iota(jnp.int32, x.shape, 0)
    cidx = lax.broadcasted_iota(jnp.int32, x.shape, 1)
    valid = (ridx * 128 + cidx) < NS
    xm = jnp.where(valid, x, -jnp.inf)
    m = jnp.max(xm)
    e = jnp.where(valid, jnp.exp(x - m), 0.0)
    o_ref[...] = e / jnp.sum(e)


def _softmax_over_all(logit):
    x = jnp.zeros((_SM_ROWS * 128,), jnp.float32).at[:NS].set(logit)
    x = x.reshape(_SM_ROWS, 128)
    y = pl.pallas_call(
        _softmax_body,
        out_shape=jax.ShapeDtypeStruct((_SM_ROWS, 128), jnp.float32),
    )(x)
    return y.reshape(-1)[:NS]


def _pad16(x):
    return jnp.pad(x, ((0, 0), (0, 16 - x.shape[1])))


_BLK = 1000
_NBLK = NS // _BLK


def _rs(w):
    return pl.BlockSpec((_BLK, w), lambda i: (i, 0))


def _fs(shape):
    return pl.BlockSpec(shape, lambda i: (0,) * len(shape))


def _accmax(i, ref, val):
    @pl.when(i == 0)
    def _():
        ref[...] = jnp.full((1, 1), val, jnp.float32)

    @pl.when(i > 0)
    def _():
        ref[...] = jnp.maximum(ref[...], val)


def _sum2_body(a_ref, b_ref, o_ref):
    o_ref[...] = a_ref[...] + b_ref[...]


def _sum2(x2):
    w = x2.shape[2]
    return pl.pallas_call(
        _sum2_body,
        grid=(_NBLK,),
        in_specs=[_rs(w), _rs(w)],
        out_specs=_rs(w),
        out_shape=jax.ShapeDtypeStruct((NS, w), jnp.float32),
    )(x2[0], x2[1])


def _dis_body(deg_ref, o_ref):
    col = deg_ref[...][:, 0:1]
    o_ref[...] = jnp.where(col > 0, lax.rsqrt(jnp.maximum(col, 1.0)), 0.0)


def _dis_kernel(deg8):
    return pl.pallas_call(
        _dis_body,
        grid=(_NBLK,),
        in_specs=[_rs(8)],
        out_specs=_rs(1),
        out_shape=jax.ShapeDtypeStruct((NG, 1), jnp.float32),
    )(deg8)[:, 0]


def _game_body(x_ref, h1_ref, h2_ref, w0_ref, w1_ref, w2_ref, b_ref, ws_ref,
               as_ref, g_ref, hs_ref, asn_ref, mx_ref):
    i = pl.program_id(0)
    out = (jnp.dot(x_ref[...], w0_ref[...])
           + jnp.dot(h1_ref[...][:, :5], w1_ref[...])
           + jnp.dot(h2_ref[...][:, :5], w2_ref[...]) + b_ref[...])
    g = jnp.maximum(out, 0.0)
    g_ref[...] = g
    hs = jnp.dot(g, ws_ref[...])
    hs_ref[...] = hs
    asn = jnp.dot(hs, as_ref[...])
    asn_ref[...] = asn
    _accmax(i, mx_ref, jnp.max(asn))


def _game_kernel(x_game, h1, h2, w0, w1, w2, b, ws, a_s):
    return pl.pallas_call(
        _game_body,
        grid=(_NBLK,),
        in_specs=[_rs(5), _rs(16), _rs(16), _fs((5, 64)), _fs((5, 64)),
                  _fs((5, 64)), _fs((1, 64)), _fs((64, 64)), _fs((64, 1))],
        out_specs=[_rs(64), _rs(64), _rs(1), pl.BlockSpec((1, 1), lambda i: (0, 0))],
        out_shape=[jax.ShapeDtypeStruct((NG, 64), jnp.float32),
                   jax.ShapeDtypeStruct((NG, 64), jnp.float32),
                   jax.ShapeDtypeStruct((NG, 1), jnp.float32),
                   jax.ShapeDtypeStruct((1, 1), jnp.float32)],
    )(x_game, h1, h2, w0, w1, w2, b[None, :], ws, a_s[:, None])


def _state_body(ssum_ref, cnt_ref, x_ref, wl_ref, wr_ref, b_ref, wdad_ref,
                s2_ref, adn_ref, mx_ref):
    i = pl.program_id(0)
    cnt = jnp.maximum(cnt_ref[...][:, 0:1], 1.0)
    mean = ssum_ref[...][:, :6] / cnt
    s2 = jnp.maximum(jnp.dot(mean, wl_ref[...])
                     + jnp.dot(x_ref[...], wr_ref[...]) + b_ref[...], 0.0)
    s2_ref[...] = s2
    adn = jnp.dot(s2, wdad_ref[...])
    adn_ref[...] = adn
    _accmax(i, mx_ref, jnp.max(adn))


def _state_kernel(s_sum, cnt8, x_state, wl, wr, b, wd_ad):
    return pl.pallas_call(
        _state_body,
        grid=(_NBLK,),
        in_specs=[_rs(16), _rs(8), _rs(6), _fs((6, 64)), _fs((6, 64)),
                  _fs((1, 64)), _fs((64, 1))],
        out_specs=[_rs(64), _rs(1), pl.BlockSpec((1, 1), lambda i: (0, 0))],
        out_shape=[jax.ShapeDtypeStruct((NS, 64), jnp.float32),
                   jax.ShapeDtypeStruct((NS, 1), jnp.float32),
                   jax.ShapeDtypeStruct((1, 1), jnp.float32)],
    )(s_sum, cnt8, x_state, wl, wr, b[None, :], wd_ad[:, None])


_EBLK = 8000


def _edge_body(ea_ref, c_ref, aen_ref, mx_ref):
    i = pl.program_id(0)
    ea = ea_ref[...]
    c = c_ref[...]
    v = ea[:, 0:1] * c[0, 0] + ea[:, 1:2] * c[0, 1]
    aen_ref[...] = v
    _accmax(i, mx_ref, jnp.max(v))


def _edge_kernel(edge_attr, we_ae):
    return pl.pallas_call(
        _edge_body,
        grid=(E // _EBLK,),
        in_specs=[pl.BlockSpec((_EBLK, 2), lambda i: (i, 0)), _fs((1, 2))],
        out_specs=[pl.BlockSpec((_EBLK, 1), lambda i: (i, 0)),
                   pl.BlockSpec((1, 1), lambda i: (0, 0))],
        out_shape=[jax.ShapeDtypeStruct((E, 1), jnp.float32),
                   jax.ShapeDtypeStruct((1, 1), jnp.float32)],
    )(edge_attr, we_ae[None, :])


def _final_body(u_ref, den_ref, isum_ref, icnt_ref, gb_ref, wl_ref, wr_ref,
                b4_ref, lw_ref, lb_ref, llw_ref, llb_ref, logit_ref):
    den = jnp.maximum(den_ref[...][:, 0:1], 1e-16)
    s3 = jnp.maximum(u_ref[...] / den + gb_ref[...], 0.0)
    icnt = jnp.maximum(icnt_ref[...][:, 0:1], 1.0)
    mean = isum_ref[...] / icnt
    s4 = jnp.maximum(jnp.dot(mean, wl_ref[...])
                     + jnp.dot(s3, wr_ref[...]) + b4_ref[...], 0.0)
    s5 = jnp.dot(s4, lw_ref[...]) + lb_ref[...]
    logit_ref[...] = jnp.dot(s5, llw_ref[...]) + llb_ref[...]


def _final_kernel(u, den16, in_sum, icnt8, gb, wl, wr, b4, lw, lb, llw, llb):
    return pl.pallas_call(
        _final_body,
        grid=(_NBLK,),
        in_specs=[_rs(64), _rs(16), _rs(64), _rs(8), _fs((1, 64)),
                  _fs((64, 64)), _fs((64, 64)), _fs((1, 64)), _fs((64, 8)),
                  _fs((1, 8)), _fs((8, 1)), _fs((1, 1))],
        out_specs=_rs(1),
        out_shape=jax.ShapeDtypeStruct((NS, 1), jnp.float32),
    )(u, den16, in_sum, icnt8, gb[None, :], wl, wr, b4[None, :], lw,
      lb[None, :], llw, llb[None, :])


def kernel(x_game, x_state, edge_attr, params, edge_index_gg, edge_index_ss, edge_index_hist, edge_index_in):
    p = params
    e2 = lambda v: v
    _Z64 = jnp.zeros((ZCH, 64), jnp.float32)
    _Z16 = jnp.zeros((ZCH, 16), jnp.float32)
    _Z8 = jnp.zeros((ZCH, 8), jnp.float32)
    _ONES8 = jnp.ones((128, 8), jnp.float32)
    fold = lambda c: c[:, 0]

    deg8 = _sum2(_seg_count(e2(edge_index_gg[1]), _Z8, _ONES8))
    dis = _dis_kernel(deg8)
    x16 = _pad16(x_game)
    h1 = _sum2(_seg_norm_16(x16, dis, e2(edge_index_gg[0]),
                            e2(edge_index_gg[1]), _Z16))
    h2 = _sum2(_seg_norm_16(h1, dis, e2(edge_index_gg[0]),
                            e2(edge_index_gg[1]), _Z16))

    s_sum2, cnt8p = _seg_plain_16(_pad16(x_state), e2(edge_index_ss[0]),
                                  e2(edge_index_ss[1]), _Z16, _Z8, _ONES8)
    s_sum, cnt8 = _sum2(s_sum2), _sum2(cnt8p)

    g, hs, asn1, asmax = _game_kernel(x_game, h1, h2, p['tag_W'][0],
                                      p['tag_W'][1], p['tag_W'][2],
                                      p['tag_b'], p['g_Ws'], p['g_as'])
    s2, adn1, admax = _state_kernel(s_sum, cnt8, x_state, p['s2_Wl'],
                                    p['s2_Wr'], p['s2_b'],
                                    p['g_Wd'] @ p['g_ad'])
    aen1, aemax = _edge_kernel(edge_attr, p['g_We'] @ p['g_ae'])

    bnd = asmax[0, 0] + admax[0, 0] + aemax[0, 0]
    bnd = jnp.maximum(bnd, 0.2 * bnd)
    bnd16 = jnp.full((16,), bnd)
    asn, adn, aen = asn1[:, 0], adn1[:, 0], aen1[:, 0]
    u = _seg_gat(hs, asn, adn, e2(aen), bnd16,
                 e2(edge_index_hist[0]), e2(edge_index_hist[1]), _Z64)
    den = _sum2(_seg_wcount(asn, adn, e2(aen), bnd16,
                            e2(edge_index_hist[0]), e2(edge_index_hist[1]),
                            _Z16))

    in_sum, icnt8 = _seg_plain_64(g, e2(edge_index_in[0]),
                                  e2(edge_index_in[1]), _Z64, _Z8, _ONES8)

    logit = _final_kernel(u, den, in_sum, icnt8, p['g_b'], p['s4_Wl'],
                          p['s4_Wr'], p['s4_b'], p['lin_W'], p['lin_b'],
                          p['ll_W'], p['ll_b'])
    state_out = _softmax_over_all(logit[:, 0])[:, None]
    return (state_out, x_game)


# speedup vs baseline: 1.4718x; 1.1601x over previous
import functools

import jax
import jax.numpy as jnp
from jax import lax
from jax.experimental import pallas as pl
from jax.experimental.pallas import tpu as pltpu
from jax.experimental.pallas import tpu_sc as plsc

NG = 50000
NS = 50000
E = 800000
HID = 64

NSUB = 16
NCH = E // 128
CH_PER_TILE = NCH // NSUB
CH_EXTRA = NCH % NSUB
HALF = NS // 2
ACC_ROWS = 25088
ZROWS_PER_TILE = ACC_ROWS // NSUB
TRASH = 25024
OUT_PER_TILE = 1568
ZCH = 224

_MESH = plsc.VectorSubcoreMesh(core_axis_name="c", subcore_axis_name="s")

ACC2_ROWS = 50176
Z2_PER_TILE = ACC2_ROWS // NSUB
OUT2_PER_TILE = 3136
NCH2 = NCH // 2
CH2_PER_TILE = NCH2 // NSUB
CH2_EXTRA = NCH2 % NSUB


def _nchunks(s_idx):
    return jnp.where(s_idx < CH_EXTRA, CH_PER_TILE + 1, CH_PER_TILE)


def _nchunks2(s_idx):
    return jnp.where(s_idx < CH2_EXTRA, CH2_PER_TILE + 1, CH2_PER_TILE)


def _compute_local_dst(dstb, ldb, base):
    def j_body(j, _):
        off = pl.multiple_of(j * 16, 16)
        d16 = dstb[pl.ds(off, 16)]
        ld = d16 - base
        ok = (ld >= 0) & (ld < HALF)
        ldb[pl.ds(off, 16)] = jnp.where(ok, ld, TRASH)
        return 0
    lax.fori_loop(0, 8, j_body, 0)


def _zero_acc(zsrc, acc, s_idx, per_tile=ZROWS_PER_TILE):
    z0 = s_idx * per_tile
    for i in range(per_tile // ZCH):
        pltpu.sync_copy(zsrc, acc.at[pl.ds(z0 + i * ZCH, ZCH)])


def _write_out(acc, out, c_idx, s_idx, fr=False):
    per = OUT2_PER_TILE if fr else OUT_PER_TILE
    tot = NS if fr else HALF

    @pl.when(s_idx < NSUB - 1)
    def _():
        r0 = pl.multiple_of(s_idx * per, 8)
        if fr:
            pltpu.sync_copy(acc.at[pl.ds(r0, per)],
                            out.at[c_idx, pl.ds(r0, per)])
        else:
            o0 = pl.multiple_of(c_idx * HALF + r0, 8)
            pltpu.sync_copy(acc.at[pl.ds(r0, per)], out.at[pl.ds(o0, per)])

    last0 = (NSUB - 1) * per
    @pl.when(s_idx == NSUB - 1)
    def _():
        if fr:
            pltpu.sync_copy(acc.at[pl.ds(last0, tot - last0)],
                            out.at[c_idx, pl.ds(last0, tot - last0)])
        else:
            o0 = pl.multiple_of(c_idx * HALF + last0, 8)
            pltpu.sync_copy(acc.at[pl.ds(last0, tot - last0)],
                            out.at[pl.ds(o0, tot - last0)])


def _scale_rows(rows, wref, w, densb=None):
    ngrp = w // 16

    def g_body(jg, _):
        off = pl.multiple_of(jg * 16, 16)
        w16 = wref[pl.ds(off, 16)]
        for r in range(16):
            sp = jnp.full((16,), w16[r], jnp.float32)
            row = jg * 16 + r
            if densb is not None:
                densb[row, :] = sp
            for j in range(ngrp):
                o2 = pl.multiple_of(j * 16, 16)
                rows[row, pl.ds(o2, 16)] = rows[row, pl.ds(o2, 16)] * sp
        return 0
    lax.fori_loop(0, 8, g_body, 0)


def _make_seg_kernel(mode, w, fr=False):
    with_rows = mode in ("plain", "norm", "gat")
    with_cnt = mode in ("plain", "count")
    cw = 8 if with_cnt else 16
    arows = ACC2_ROWS if fr else ACC_ROWS
    oshape = (lambda ww: (2, NS, ww)) if fr else (lambda ww: (NS, ww))

    out_type = []
    if with_rows:
        out_type.append(jax.ShapeDtypeStruct(oshape(w), jnp.float32))
    if with_cnt or mode == "wcount":
        out_type.append(jax.ShapeDtypeStruct(oshape(cw), jnp.float32))

    scratch = [pltpu.VMEM((128,), jnp.int32)] * 2
    scratch += [pltpu.VMEM((128,), jnp.int32)] * 2
    if with_rows or mode == "wcount":
        scratch += [pltpu.VMEM((128,), jnp.int32)] * 2
    if with_rows:
        scratch += [pltpu.VMEM((128, w), jnp.float32)] * 2
        scratch.append(pltpu.VMEM_SHARED((arows, w), jnp.float32))
    if with_cnt:
        scratch.append(pltpu.VMEM_SHARED((arows, 8), jnp.float32))
        scratch.append(pltpu.VMEM((128, 8), jnp.float32))
    if mode == "wcount":
        scratch.append(pltpu.VMEM_SHARED((arows, 16), jnp.float32))
        scratch.append(pltpu.VMEM((128, 16), jnp.float32))
    if mode == "norm":
        scratch += [pltpu.VMEM((128,), jnp.float32)] * 2
        scratch += [pltpu.VMEM((128,), jnp.float32)] * 2
        scratch.append(pltpu.VMEM((128,), jnp.float32))
    if mode in ("gat", "wcount"):
        scratch += [pltpu.VMEM((128,), jnp.float32)] * 2
        scratch += [pltpu.VMEM((128,), jnp.float32)] * 2
        scratch += [pltpu.VMEM((128,), jnp.float32)] * 2
        scratch.append(pltpu.VMEM((128,), jnp.float32))
        scratch.append(pltpu.VMEM((16,), jnp.float32))
    scratch += [pltpu.SemaphoreType.DMA] * 6

    def body(*refs):
        it = iter(refs)
        if with_rows:
            table = next(it)
        if mode == "norm":
            wnode = next(it)
        if mode in ("gat", "wcount"):
            asn, adn, aef, bnd = next(it), next(it), next(it), next(it)
        srcf = next(it) if (with_rows or mode == "wcount") else None
        dstf = next(it)
        zrows = next(it) if with_rows else None
        zcnt = next(it) if (with_cnt or mode == "wcount") else None
        ones8 = next(it) if with_cnt else None
        out_rows = next(it) if with_rows else None
        out_cnt = next(it) if (with_cnt or mode == "wcount") else None
        dstb = [next(it), next(it)]
        ldb = [next(it), next(it)]
        srcb = ([next(it), next(it)]
                if (with_rows or mode == "wcount") else [None, None])
        if with_rows:
            rows = [next(it), next(it)]
            acc = next(it)
        if with_cnt:
            cacc = next(it)
            onesb = next(it)
        if mode == "wcount":
            cacc = next(it)
            densb = next(it)
        if mode == "norm":
            wsv = [next(it), next(it)]
            wdv = [next(it), next(it)]
            wb = next(it)
        if mode in ("gat", "wcount"):
            asv = [next(it), next(it)]
            adv = [next(it), next(it)]
            aev = [next(it), next(it)]
            exb = next(it)
            bndv = next(it)
        sem = [next(it), next(it)]
        sem2 = [next(it), next(it)]
        sem3 = [next(it), next(it)]

        c_idx = lax.axis_index("c")
        s_idx = lax.axis_index("s")
        base = c_idx * HALF
        zper = Z2_PER_TILE if fr else ZROWS_PER_TILE

        if with_rows:
            _zero_acc(zrows, acc, s_idx, zper)
        if with_cnt or mode == "wcount":
            _zero_acc(zcnt, cacc, s_idx, zper)
        if with_cnt:
            pltpu.sync_copy(ones8, onesb)
        if mode in ("gat", "wcount"):
            pltpu.sync_copy(bnd, bndv)
        plsc.subcore_barrier()

        def stage_load(k, b):
            if fr:
                chunk = c_idx * NCH2 + s_idx + NSUB * k
            else:
                chunk = s_idx + NSUB * k
            e0 = pl.multiple_of(chunk * 128, 128)
            pltpu.sync_copy(dstf.at[pl.ds(e0, 128)], dstb[b])
            if srcb[b] is not None:
                pltpu.sync_copy(srcf.at[pl.ds(e0, 128)], srcb[b])
            if mode in ("gat", "wcount"):
                pltpu.sync_copy(aef.at[pl.ds(e0, 128)], aev[b])
            if not fr:
                _compute_local_dst(dstb[b], ldb[b], base)

            hs = []
            if with_rows:
                hs.append(pltpu.async_copy(table.at[srcb[b]], rows[b], sem[b]))
            if mode == "norm":
                hs.append(pltpu.async_copy(wnode.at[srcb[b]], wsv[b], sem2[b]))
                hs.append(pltpu.async_copy(wnode.at[dstb[b]], wdv[b], sem3[b]))
            if mode in ("gat", "wcount"):
                hs.append(pltpu.async_copy(asn.at[srcb[b]], asv[b], sem2[b]))
                hs.append(pltpu.async_copy(adn.at[dstb[b]], adv[b], sem3[b]))
            return hs

        def stage_exec(b, hs):
            for h in hs:
                h.wait()
            sidx = dstb[b] if fr else ldb[b]

            if mode == "norm":
                def jn_body(j, _):
                    off = pl.multiple_of(j * 16, 16)
                    wb[pl.ds(off, 16)] = (wsv[b][pl.ds(off, 16)]
                                          * wdv[b][pl.ds(off, 16)])
                    return 0
                lax.fori_loop(0, 8, jn_body, 0)
                _scale_rows(rows[b], wb, w)
            if mode in ("gat", "wcount"):
                bb = bndv[...]
                def jg_body(j, _):
                    off = pl.multiple_of(j * 16, 16)
                    t = (asv[b][pl.ds(off, 16)] + adv[b][pl.ds(off, 16)]
                         + aev[b][pl.ds(off, 16)])
                    t = jnp.maximum(t, 0.2 * t)
                    exb[pl.ds(off, 16)] = jnp.exp(t - bb)
                    return 0
                lax.fori_loop(0, 8, jg_body, 0)
            if mode == "gat":
                _scale_rows(rows[b], exb, w)
            if mode == "wcount":
                def js_body(jg, _):
                    off = pl.multiple_of(jg * 16, 16)
                    w16 = exb[pl.ds(off, 16)]
                    for r in range(16):
                        densb[jg * 16 + r, :] = jnp.full((16,), w16[r], jnp.float32)
                    return 0
                lax.fori_loop(0, 8, js_body, 0)

            if with_rows:
                pltpu.sync_copy(rows[b], acc.at[sidx], add=True)
            if with_cnt:
                pltpu.sync_copy(onesb, cacc.at[sidx], add=True)
            if mode == "wcount":
                pltpu.sync_copy(densb, cacc.at[sidx], add=True)

        def pair_body(kp, _):
            ha = stage_load(2 * kp, 0)
            hb = stage_load(2 * kp + 1, 1)
            stage_exec(0, ha)
            stage_exec(1, hb)
            return 0

        nk = _nchunks2(s_idx) if fr else _nchunks(s_idx)
        lax.fori_loop(0, nk // 2, pair_body, 0)

        @pl.when(nk % 2 == 1)
        def _():
            stage_exec(0, stage_load(nk - 1, 0))
        plsc.subcore_barrier()
        if with_rows:
            _write_out(acc, out_rows, c_idx, s_idx, fr)
        if with_cnt or mode == "wcount":
            _write_out(cacc, out_cnt, c_idx, s_idx, fr)

    if len(out_type) == 1:
        out_type = out_type[0]
    return functools.partial(
        pl.kernel, mesh=_MESH, out_type=out_type, scratch_types=scratch,
        compiler_params=pltpu.CompilerParams(use_tc_tiling_on_sc=False),
    )(body)


_seg_plain_64 = _make_seg_kernel("plain", 64)
_seg_plain_16 = _make_seg_kernel("plain", 16, fr=True)
_seg_norm_16 = _make_seg_kernel("norm", 16, fr=True)
_seg_gat = _make_seg_kernel("gat", 64)
_seg_count = _make_seg_kernel("count", 0, fr=True)
_seg_wcount = _make_seg_kernel("wcount", 0, fr=True)

_SM_ROWS = 392


def _softmax_body(x_ref, o_ref):
    x = x_ref[...]
    ridx = lax.broadcasted_iota(jnp.int32, x.shape, 0)
    cidx = lax.broadcasted_iota(jnp.int32, x.shape, 1)
    valid = (ridx * 128 + cidx) < NS
    xm = jnp.where(valid, x, -jnp.inf)
    m = jnp.max(xm)
    e = jnp.where(valid, jnp.exp(x - m), 0.0)
    o_ref[...] = e / jnp.sum(e)


def _softmax_over_all(logit):
    x = jnp.zeros((_SM_ROWS * 128,), jnp.float32).at[:NS].set(logit)
    x = x.reshape(_SM_ROWS, 128)
    y = pl.pallas_call(
        _softmax_body,
        out_shape=jax.ShapeDtypeStruct((_SM_ROWS, 128), jnp.float32),
    )(x)
    return y.reshape(-1)[:NS]


def _pad16(x):
    return jnp.pad(x, ((0, 0), (0, 16 - x.shape[1])))


_BLK = 1000
_NBLK = NS // _BLK


def _rs(w):
    return pl.BlockSpec((_BLK, w), lambda i: (i, 0))


def _fs(shape):
    return pl.BlockSpec(shape, lambda i: (0,) * len(shape))


def _accmax(i, ref, val):
    @pl.when(i == 0)
    def _():
        ref[...] = jnp.full((1, 1), val, jnp.float32)

    @pl.when(i > 0)
    def _():
        ref[...] = jnp.maximum(ref[...], val)


def _sum2_body(a_ref, b_ref, o_ref):
    o_ref[...] = a_ref[...] + b_ref[...]


def _sum2(x2):
    w = x2.shape[2]
    return pl.pallas_call(
        _sum2_body,
        grid=(_NBLK,),
        in_specs=[_rs(w), _rs(w)],
        out_specs=_rs(w),
        out_shape=jax.ShapeDtypeStruct((NS, w), jnp.float32),
    )(x2[0], x2[1])


def _dis_body(deg_ref, o_ref):
    col = deg_ref[...][:, 0:1]
    o_ref[...] = jnp.where(col > 0, lax.rsqrt(jnp.maximum(col, 1.0)), 0.0)


def _dis_kernel(deg8):
    return pl.pallas_call(
        _dis_body,
        grid=(_NBLK,),
        in_specs=[_rs(8)],
        out_specs=_rs(1),
        out_shape=jax.ShapeDtypeStruct((NG, 1), jnp.float32),
    )(deg8)[:, 0]


def _game_body(x_ref, h1_ref, h2_ref, w0_ref, w1_ref, w2_ref, b_ref, ws_ref,
               as_ref, g_ref, hs_ref, asn_ref, mx_ref):
    i = pl.program_id(0)
    out = (jnp.dot(x_ref[...], w0_ref[...])
           + jnp.dot(h1_ref[...][:, :5], w1_ref[...])
           + jnp.dot(h2_ref[...][:, :5], w2_ref[...]) + b_ref[...])
    g = jnp.maximum(out, 0.0)
    g_ref[...] = g
    hs = jnp.dot(g, ws_ref[...])
    hs_ref[...] = hs
    asn = jnp.dot(hs, as_ref[...])
    asn_ref[...] = asn
    _accmax(i, mx_ref, jnp.max(asn))


def _game_kernel(x_game, h1, h2, w0, w1, w2, b, ws, a_s):
    return pl.pallas_call(
        _game_body,
        grid=(_NBLK,),
        in_specs=[_rs(5), _rs(16), _rs(16), _fs((5, 64)), _fs((5, 64)),
                  _fs((5, 64)), _fs((1, 64)), _fs((64, 64)), _fs((64, 1))],
        out_specs=[_rs(64), _rs(64), _rs(1), pl.BlockSpec((1, 1), lambda i: (0, 0))],
        out_shape=[jax.ShapeDtypeStruct((NG, 64), jnp.float32),
                   jax.ShapeDtypeStruct((NG, 64), jnp.float32),
                   jax.ShapeDtypeStruct((NG, 1), jnp.float32),
                   jax.ShapeDtypeStruct((1, 1), jnp.float32)],
    )(x_game, h1, h2, w0, w1, w2, b[None, :], ws, a_s[:, None])


def _state_body(ssum_ref, cnt_ref, x_ref, wl_ref, wr_ref, b_ref, wdad_ref,
                s2_ref, adn_ref, mx_ref):
    i = pl.program_id(0)
    cnt = jnp.maximum(cnt_ref[...][:, 0:1], 1.0)
    mean = ssum_ref[...][:, :6] / cnt
    s2 = jnp.maximum(jnp.dot(mean, wl_ref[...])
                     + jnp.dot(x_ref[...], wr_ref[...]) + b_ref[...], 0.0)
    s2_ref[...] = s2
    adn = jnp.dot(s2, wdad_ref[...])
    adn_ref[...] = adn
    _accmax(i, mx_ref, jnp.max(adn))


def _state_kernel(s_sum, cnt8, x_state, wl, wr, b, wd_ad):
    return pl.pallas_call(
        _state_body,
        grid=(_NBLK,),
        in_specs=[_rs(16), _rs(8), _rs(6), _fs((6, 64)), _fs((6, 64)),
                  _fs((1, 64)), _fs((64, 1))],
        out_specs=[_rs(64), _rs(1), pl.BlockSpec((1, 1), lambda i: (0, 0))],
        out_shape=[jax.ShapeDtypeStruct((NS, 64), jnp.float32),
                   jax.ShapeDtypeStruct((NS, 1), jnp.float32),
                   jax.ShapeDtypeStruct((1, 1), jnp.float32)],
    )(s_sum, cnt8, x_state, wl, wr, b[None, :], wd_ad[:, None])


_EBLK = 8000


def _edge_body(ea_ref, c_ref, aen_ref, mx_ref):
    i = pl.program_id(0)
    ea = ea_ref[...]
    c = c_ref[...]
    v = ea[:, 0:1] * c[0, 0] + ea[:, 1:2] * c[0, 1]
    aen_ref[...] = v
    _accmax(i, mx_ref, jnp.max(v))


def _edge_kernel(edge_attr, we_ae):
    return pl.pallas_call(
        _edge_body,
        grid=(E // _EBLK,),
        in_specs=[pl.BlockSpec((_EBLK, 2), lambda i: (i, 0)), _fs((1, 2))],
        out_specs=[pl.BlockSpec((_EBLK, 1), lambda i: (i, 0)),
                   pl.BlockSpec((1, 1), lambda i: (0, 0))],
        out_shape=[jax.ShapeDtypeStruct((E, 1), jnp.float32),
                   jax.ShapeDtypeStruct((1, 1), jnp.float32)],
    )(edge_attr, we_ae[None, :])


def _final_body(u_ref, den_ref, isum_ref, icnt_ref, gb_ref, wl_ref, wr_ref,
                b4_ref, lw_ref, lb_ref, llw_ref, llb_ref, logit_ref):
    den = jnp.maximum(den_ref[...][:, 0:1], 1e-16)
    s3 = jnp.maximum(u_ref[...] / den + gb_ref[...], 0.0)
    icnt = jnp.maximum(icnt_ref[...][:, 0:1], 1.0)
    mean = isum_ref[...] / icnt
    s4 = jnp.maximum(jnp.dot(mean, wl_ref[...])
                     + jnp.dot(s3, wr_ref[...]) + b4_ref[...], 0.0)
    s5 = jnp.dot(s4, lw_ref[...]) + lb_ref[...]
    logit_ref[...] = jnp.dot(s5, llw_ref[...]) + llb_ref[...]


def _final_kernel(u, den16, in_sum, icnt8, gb, wl, wr, b4, lw, lb, llw, llb):
    return pl.pallas_call(
        _final_body,
        grid=(_NBLK,),
        in_specs=[_rs(64), _rs(16), _rs(64), _rs(8), _fs((1, 64)),
                  _fs((64, 64)), _fs((64, 64)), _fs((1, 64)), _fs((64, 8)),
                  _fs((1, 8)), _fs((8, 1)), _fs((1, 1))],
        out_specs=_rs(1),
        out_shape=jax.ShapeDtypeStruct((NS, 1), jnp.float32),
    )(u, den16, in_sum, icnt8, gb[None, :], wl, wr, b4[None, :], lw,
      lb[None, :], llw, llb[None, :])


def kernel(x_game, x_state, edge_attr, params, edge_index_gg, edge_index_ss, edge_index_hist, edge_index_in):
    p = params
    e2 = lambda v: v
    _Z64 = jnp.zeros((ZCH, 64), jnp.float32)
    _Z16 = jnp.zeros((ZCH, 16), jnp.float32)
    _Z8 = jnp.zeros((ZCH, 8), jnp.float32)
    _ONES8 = jnp.ones((128, 8), jnp.float32)
    fold = lambda c: c[:, 0]

    deg8 = _sum2(_seg_count(e2(edge_index_gg[1]), _Z8, _ONES8))
    dis = _dis_kernel(deg8)
    x16 = _pad16(x_game)
    h1 = _sum2(_seg_norm_16(x16, dis, e2(edge_index_gg[0]),
                            e2(edge_index_gg[1]), _Z16))
    h2 = _sum2(_seg_norm_16(h1, dis, e2(edge_index_gg[0]),
                            e2(edge_index_gg[1]), _Z16))

    s_sum2, cnt8p = _seg_plain_16(_pad16(x_state), e2(edge_index_ss[0]),
                                  e2(edge_index_ss[1]), _Z16, _Z8, _ONES8)
    s_sum, cnt8 = _sum2(s_sum2), _sum2(cnt8p)

    g, hs, asn1, asmax = _game_kernel(x_game, h1, h2, p['tag_W'][0],
                                      p['tag_W'][1], p['tag_W'][2],
                                      p['tag_b'], p['g_Ws'], p['g_as'])
    s2, adn1, admax = _state_kernel(s_sum, cnt8, x_state, p['s2_Wl'],
                                    p['s2_Wr'], p['s2_b'],
                                    p['g_Wd'] @ p['g_ad'])
    aen1, aemax = _edge_kernel(edge_attr, p['g_We'] @ p['g_ae'])

    bnd = asmax[0, 0] + admax[0, 0] + aemax[0, 0]
    bnd = jnp.maximum(bnd, 0.2 * bnd)
    bnd16 = jnp.full((16,), bnd)
    asn, adn, aen = asn1[:, 0], adn1[:, 0], aen1[:, 0]
    u = _seg_gat(hs, asn, adn, e2(aen), bnd16,
                 e2(edge_index_hist[0]), e2(edge_index_hist[1]), _Z64)
    den = _sum2(_seg_wcount(asn, adn, e2(aen), bnd16,
                            e2(edge_index_hist[0]), e2(edge_index_hist[1]),
                            _Z16))

    in_sum, icnt8 = _seg_plain_64(g, e2(edge_index_in[0]),
                                  e2(edge_index_in[1]), _Z64, _Z8, _ONES8)

    logit = _final_kernel(u, den, in_sum, icnt8, p['g_b'], p['s4_Wl'],
                          p['s4_Wr'], p['s4_b'], p['lin_W'], p['lin_b'],
                          p['ll_W'], p['ll_b'])
    state_out = _softmax_over_all(logit[:, 0])[:, None]
    return (state_out, x_game)


# speedup vs baseline: 1.4921x; 1.0138x over previous
import functools

import jax
import jax.numpy as jnp
from jax import lax
from jax.experimental import pallas as pl
from jax.experimental.pallas import tpu as pltpu
from jax.experimental.pallas import tpu_sc as plsc

NG = 50000
NS = 50000
E = 800000
HID = 64

NSUB = 16
NCH = E // 128
CH_PER_TILE = NCH // NSUB
CH_EXTRA = NCH % NSUB
HALF = NS // 2
ACC_ROWS = 25088
ZROWS_PER_TILE = ACC_ROWS // NSUB
TRASH = 25024
OUT_PER_TILE = 1568
ZCH = 224

_MESH = plsc.VectorSubcoreMesh(core_axis_name="c", subcore_axis_name="s")

ACC2_ROWS = 50176
Z2_PER_TILE = ACC2_ROWS // NSUB
OUT2_PER_TILE = 3136
NCH2 = NCH // 2
CH2_PER_TILE = NCH2 // NSUB
CH2_EXTRA = NCH2 % NSUB


def _nchunks(s_idx):
    return jnp.where(s_idx < CH_EXTRA, CH_PER_TILE + 1, CH_PER_TILE)


def _nchunks2(s_idx):
    return jnp.where(s_idx < CH2_EXTRA, CH2_PER_TILE + 1, CH2_PER_TILE)


def _compute_local_dst(dstb, ldb, base):
    def j_body(j, _):
        off = pl.multiple_of(j * 16, 16)
        d16 = dstb[pl.ds(off, 16)]
        ld = d16 - base
        ok = (ld >= 0) & (ld < HALF)
        ldb[pl.ds(off, 16)] = jnp.where(ok, ld, TRASH)
        return 0
    lax.fori_loop(0, 8, j_body, 0)


def _zero_acc(zsrc, acc, s_idx, per_tile=ZROWS_PER_TILE):
    z0 = s_idx * per_tile
    for i in range(per_tile // ZCH):
        pltpu.sync_copy(zsrc, acc.at[pl.ds(z0 + i * ZCH, ZCH)])


def _write_out(acc, out, c_idx, s_idx, fr=False):
    per = OUT2_PER_TILE if fr else OUT_PER_TILE
    tot = NS if fr else HALF

    @pl.when(s_idx < NSUB - 1)
    def _():
        r0 = pl.multiple_of(s_idx * per, 8)
        if fr:
            pltpu.sync_copy(acc.at[pl.ds(r0, per)],
                            out.at[c_idx, pl.ds(r0, per)])
        else:
            o0 = pl.multiple_of(c_idx * HALF + r0, 8)
            pltpu.sync_copy(acc.at[pl.ds(r0, per)], out.at[pl.ds(o0, per)])

    last0 = (NSUB - 1) * per
    @pl.when(s_idx == NSUB - 1)
    def _():
        if fr:
            pltpu.sync_copy(acc.at[pl.ds(last0, tot - last0)],
                            out.at[c_idx, pl.ds(last0, tot - last0)])
        else:
            o0 = pl.multiple_of(c_idx * HALF + last0, 8)
            pltpu.sync_copy(acc.at[pl.ds(last0, tot - last0)],
                            out.at[pl.ds(o0, tot - last0)])


def _scale_rows(rows, wref, w, densb=None):
    ngrp = w // 16

    def g_body(jg, _):
        off = pl.multiple_of(jg * 16, 16)
        w16 = wref[pl.ds(off, 16)]
        for r in range(16):
            sp = jnp.full((16,), w16[r], jnp.float32)
            row = jg * 16 + r
            if densb is not None:
                densb[row, :] = sp
            for j in range(ngrp):
                o2 = pl.multiple_of(j * 16, 16)
                rows[row, pl.ds(o2, 16)] = rows[row, pl.ds(o2, 16)] * sp
        return 0
    lax.fori_loop(0, 8, g_body, 0)


def _make_seg_kernel(mode, w, fr=False, nd=2):
    with_rows = mode in ("plain", "norm", "gat")
    with_cnt = mode in ("plain", "count")
    cw = 8 if with_cnt else 16
    arows = ACC2_ROWS if fr else ACC_ROWS
    oshape = (lambda ww: (2, NS, ww)) if fr else (lambda ww: (NS, ww))

    out_type = []
    if with_rows:
        out_type.append(jax.ShapeDtypeStruct(oshape(w), jnp.float32))
    if with_cnt or mode == "wcount":
        out_type.append(jax.ShapeDtypeStruct(oshape(cw), jnp.float32))

    scratch = [pltpu.VMEM((128,), jnp.int32)] * nd
    scratch += [pltpu.VMEM((128,), jnp.int32)] * nd
    if with_rows or mode == "wcount":
        scratch += [pltpu.VMEM((128,), jnp.int32)] * nd
    if with_rows:
        scratch += [pltpu.VMEM((128, w), jnp.float32)] * nd
        scratch.append(pltpu.VMEM_SHARED((arows, w), jnp.float32))
    if with_cnt:
        scratch.append(pltpu.VMEM_SHARED((arows, 8), jnp.float32))
        scratch.append(pltpu.VMEM((128, 8), jnp.float32))
    if mode == "wcount":
        scratch.append(pltpu.VMEM_SHARED((arows, 16), jnp.float32))
        scratch.append(pltpu.VMEM((128, 16), jnp.float32))
    if mode == "norm":
        scratch += [pltpu.VMEM((128,), jnp.float32)] * nd
        scratch += [pltpu.VMEM((128,), jnp.float32)] * nd
        scratch.append(pltpu.VMEM((128,), jnp.float32))
    if mode in ("gat", "wcount"):
        scratch += [pltpu.VMEM((128,), jnp.float32)] * nd
        scratch += [pltpu.VMEM((128,), jnp.float32)] * nd
        scratch += [pltpu.VMEM((128,), jnp.float32)] * nd
        scratch.append(pltpu.VMEM((128,), jnp.float32))
        scratch.append(pltpu.VMEM((16,), jnp.float32))
    scratch += [pltpu.SemaphoreType.DMA] * (3 * nd)

    def body(*refs):
        it = iter(refs)
        if with_rows:
            table = next(it)
        if mode == "norm":
            wnode = next(it)
        if mode in ("gat", "wcount"):
            asn, adn, aef, bnd = next(it), next(it), next(it), next(it)
        srcf = next(it) if (with_rows or mode == "wcount") else None
        dstf = next(it)
        zrows = next(it) if with_rows else None
        zcnt = next(it) if (with_cnt or mode == "wcount") else None
        ones8 = next(it) if with_cnt else None
        out_rows = next(it) if with_rows else None
        out_cnt = next(it) if (with_cnt or mode == "wcount") else None
        take = lambda n: [next(it) for _ in range(n)]
        dstb = take(nd)
        ldb = take(nd)
        srcb = take(nd) if (with_rows or mode == "wcount") else [None] * nd
        if with_rows:
            rows = take(nd)
            acc = next(it)
        if with_cnt:
            cacc = next(it)
            onesb = next(it)
        if mode == "wcount":
            cacc = next(it)
            densb = next(it)
        if mode == "norm":
            wsv = take(nd)
            wdv = take(nd)
            wb = next(it)
        if mode in ("gat", "wcount"):
            asv = take(nd)
            adv = take(nd)
            aev = take(nd)
            exb = next(it)
            bndv = next(it)
        sem = take(nd)
        sem2 = take(nd)
        sem3 = take(nd)

        c_idx = lax.axis_index("c")
        s_idx = lax.axis_index("s")
        base = c_idx * HALF
        zper = Z2_PER_TILE if fr else ZROWS_PER_TILE

        if with_rows:
            _zero_acc(zrows, acc, s_idx, zper)
        if with_cnt or mode == "wcount":
            _zero_acc(zcnt, cacc, s_idx, zper)
        if with_cnt:
            pltpu.sync_copy(ones8, onesb)
        if mode in ("gat", "wcount"):
            pltpu.sync_copy(bnd, bndv)
        plsc.subcore_barrier()

        def stage_load(k, b):
            if fr:
                chunk = c_idx * NCH2 + s_idx + NSUB * k
            else:
                chunk = s_idx + NSUB * k
            e0 = pl.multiple_of(chunk * 128, 128)
            pltpu.sync_copy(dstf.at[pl.ds(e0, 128)], dstb[b])
            if srcb[b] is not None:
                pltpu.sync_copy(srcf.at[pl.ds(e0, 128)], srcb[b])
            if mode in ("gat", "wcount"):
                pltpu.sync_copy(aef.at[pl.ds(e0, 128)], aev[b])
            if not fr:
                _compute_local_dst(dstb[b], ldb[b], base)

            hs = []
            if with_rows:
                hs.append(pltpu.async_copy(table.at[srcb[b]], rows[b], sem[b]))
            if mode == "norm":
                hs.append(pltpu.async_copy(wnode.at[srcb[b]], wsv[b], sem2[b]))
                hs.append(pltpu.async_copy(wnode.at[dstb[b]], wdv[b], sem3[b]))
            if mode in ("gat", "wcount"):
                hs.append(pltpu.async_copy(asn.at[srcb[b]], asv[b], sem2[b]))
                hs.append(pltpu.async_copy(adn.at[dstb[b]], adv[b], sem3[b]))
            return hs

        def stage_exec(b, hs):
            for h in hs:
                h.wait()
            sidx = dstb[b] if fr else ldb[b]

            if mode == "norm":
                def jn_body(j, _):
                    off = pl.multiple_of(j * 16, 16)
                    wb[pl.ds(off, 16)] = (wsv[b][pl.ds(off, 16)]
                                          * wdv[b][pl.ds(off, 16)])
                    return 0
                lax.fori_loop(0, 8, jn_body, 0)
                _scale_rows(rows[b], wb, w)
            if mode in ("gat", "wcount"):
                bb = bndv[...]
                def jg_body(j, _):
                    off = pl.multiple_of(j * 16, 16)
                    t = (asv[b][pl.ds(off, 16)] + adv[b][pl.ds(off, 16)]
                         + aev[b][pl.ds(off, 16)])
                    t = jnp.maximum(t, 0.2 * t)
                    exb[pl.ds(off, 16)] = jnp.exp(t - bb)
                    return 0
                lax.fori_loop(0, 8, jg_body, 0)
            if mode == "gat":
                _scale_rows(rows[b], exb, w)
            if mode == "wcount":
                def js_body(jg, _):
                    off = pl.multiple_of(jg * 16, 16)
                    w16 = exb[pl.ds(off, 16)]
                    for r in range(16):
                        densb[jg * 16 + r, :] = jnp.full((16,), w16[r], jnp.float32)
                    return 0
                lax.fori_loop(0, 8, js_body, 0)

            if with_rows:
                pltpu.sync_copy(rows[b], acc.at[sidx], add=True)
            if with_cnt:
                pltpu.sync_copy(onesb, cacc.at[sidx], add=True)
            if mode == "wcount":
                pltpu.sync_copy(densb, cacc.at[sidx], add=True)

        def quad_body(kp, _):
            hs4 = [stage_load(nd * kp + b, b) for b in range(nd)]
            for b in range(nd):
                stage_exec(b, hs4[b])
            return 0

        def single_body(k, _):
            stage_exec(0, stage_load(k, 0))
            return 0

        nk = _nchunks2(s_idx) if fr else _nchunks(s_idx)
        nq = nk // nd
        lax.fori_loop(0, nq, quad_body, 0)
        lax.fori_loop(nd * nq, nk, single_body, 0)
        plsc.subcore_barrier()
        if with_rows:
            _write_out(acc, out_rows, c_idx, s_idx, fr)
        if with_cnt or mode == "wcount":
            _write_out(cacc, out_cnt, c_idx, s_idx, fr)

    if len(out_type) == 1:
        out_type = out_type[0]
    return functools.partial(
        pl.kernel, mesh=_MESH, out_type=out_type, scratch_types=scratch,
        compiler_params=pltpu.CompilerParams(use_tc_tiling_on_sc=False),
    )(body)


_seg_plain_64 = _make_seg_kernel("plain", 64, nd=2)
_seg_plain_16 = _make_seg_kernel("plain", 16, fr=True, nd=4)
_seg_norm_16 = _make_seg_kernel("norm", 16, fr=True, nd=4)
_seg_gat = _make_seg_kernel("gat", 64, nd=2)
_seg_count = _make_seg_kernel("count", 0, fr=True, nd=4)
_seg_wcount = _make_seg_kernel("wcount", 0, fr=True, nd=4)

_SM_ROWS = 392


def _softmax_body(x_ref, o_ref):
    x = x_ref[...]
    ridx = lax.broadcasted_iota(jnp.int32, x.shape, 0)
    cidx = lax.broadcasted_iota(jnp.int32, x.shape, 1)
    valid = (ridx * 128 + cidx) < NS
    xm = jnp.where(valid, x, -jnp.inf)
    m = jnp.max(xm)
    e = jnp.where(valid, jnp.exp(x - m), 0.0)
    o_ref[...] = e / jnp.sum(e)


def _softmax_over_all(logit):
    x = jnp.zeros((_SM_ROWS * 128,), jnp.float32).at[:NS].set(logit)
    x = x.reshape(_SM_ROWS, 128)
    y = pl.pallas_call(
        _softmax_body,
        out_shape=jax.ShapeDtypeStruct((_SM_ROWS, 128), jnp.float32),
    )(x)
    return y.reshape(-1)[:NS]


def _pad16(x):
    return jnp.pad(x, ((0, 0), (0, 16 - x.shape[1])))


_BLK = 1000
_NBLK = NS // _BLK


def _rs(w):
    return pl.BlockSpec((_BLK, w), lambda i: (i, 0))


def _fs(shape):
    return pl.BlockSpec(shape, lambda i: (0,) * len(shape))


def _accmax(i, ref, val):
    @pl.when(i == 0)
    def _():
        ref[...] = jnp.full((1, 1), val, jnp.float32)

    @pl.when(i > 0)
    def _():
        ref[...] = jnp.maximum(ref[...], val)


def _sum2_body(a_ref, b_ref, o_ref):
    o_ref[...] = a_ref[...] + b_ref[...]


def _sum2(x2):
    w = x2.shape[2]
    return pl.pallas_call(
        _sum2_body,
        grid=(_NBLK,),
        in_specs=[_rs(w), _rs(w)],
        out_specs=_rs(w),
        out_shape=jax.ShapeDtypeStruct((NS, w), jnp.float32),
    )(x2[0], x2[1])


def _dis_body(deg_ref, o_ref):
    col = deg_ref[...][:, 0:1]
    o_ref[...] = jnp.where(col > 0, lax.rsqrt(jnp.maximum(col, 1.0)), 0.0)


def _dis_kernel(deg8):
    return pl.pallas_call(
        _dis_body,
        grid=(_NBLK,),
        in_specs=[_rs(8)],
        out_specs=_rs(1),
        out_shape=jax.ShapeDtypeStruct((NG, 1), jnp.float32),
    )(deg8)[:, 0]


def _game_body(x_ref, h1_ref, h2_ref, w0_ref, w1_ref, w2_ref, b_ref, ws_ref,
               as_ref, g_ref, hs_ref, asn_ref, mx_ref):
    i = pl.program_id(0)
    out = (jnp.dot(x_ref[...], w0_ref[...])
           + jnp.dot(h1_ref[...][:, :5], w1_ref[...])
           + jnp.dot(h2_ref[...][:, :5], w2_ref[...]) + b_ref[...])
    g = jnp.maximum(out, 0.0)
    g_ref[...] = g
    hs = jnp.dot(g, ws_ref[...])
    hs_ref[...] = hs
    asn = jnp.dot(hs, as_ref[...])
    asn_ref[...] = asn
    _accmax(i, mx_ref, jnp.max(asn))


def _game_kernel(x_game, h1, h2, w0, w1, w2, b, ws, a_s):
    return pl.pallas_call(
        _game_body,
        grid=(_NBLK,),
        in_specs=[_rs(5), _rs(16), _rs(16), _fs((5, 64)), _fs((5, 64)),
                  _fs((5, 64)), _fs((1, 64)), _fs((64, 64)), _fs((64, 1))],
        out_specs=[_rs(64), _rs(64), _rs(1), pl.BlockSpec((1, 1), lambda i: (0, 0))],
        out_shape=[jax.ShapeDtypeStruct((NG, 64), jnp.float32),
                   jax.ShapeDtypeStruct((NG, 64), jnp.float32),
                   jax.ShapeDtypeStruct((NG, 1), jnp.float32),
                   jax.ShapeDtypeStruct((1, 1), jnp.float32)],
    )(x_game, h1, h2, w0, w1, w2, b[None, :], ws, a_s[:, None])


def _state_body(ssum_ref, cnt_ref, x_ref, wl_ref, wr_ref, b_ref, wdad_ref,
                s2_ref, adn_ref, mx_ref):
    i = pl.program_id(0)
    cnt = jnp.maximum(cnt_ref[...][:, 0:1], 1.0)
    mean = ssum_ref[...][:, :6] / cnt
    s2 = jnp.maximum(jnp.dot(mean, wl_ref[...])
                     + jnp.dot(x_ref[...], wr_ref[...]) + b_ref[...], 0.0)
    s2_ref[...] = s2
    adn = jnp.dot(s2, wdad_ref[...])
    adn_ref[...] = adn
    _accmax(i, mx_ref, jnp.max(adn))


def _state_kernel(s_sum, cnt8, x_state, wl, wr, b, wd_ad):
    return pl.pallas_call(
        _state_body,
        grid=(_NBLK,),
        in_specs=[_rs(16), _rs(8), _rs(6), _fs((6, 64)), _fs((6, 64)),
                  _fs((1, 64)), _fs((64, 1))],
        out_specs=[_rs(64), _rs(1), pl.BlockSpec((1, 1), lambda i: (0, 0))],
        out_shape=[jax.ShapeDtypeStruct((NS, 64), jnp.float32),
                   jax.ShapeDtypeStruct((NS, 1), jnp.float32),
                   jax.ShapeDtypeStruct((1, 1), jnp.float32)],
    )(s_sum, cnt8, x_state, wl, wr, b[None, :], wd_ad[:, None])


_EBLK = 8000


def _edge_body(ea_ref, c_ref, aen_ref, mx_ref):
    i = pl.program_id(0)
    ea = ea_ref[...]
    c = c_ref[...]
    v = ea[:, 0:1] * c[0, 0] + ea[:, 1:2] * c[0, 1]
    aen_ref[...] = v
    _accmax(i, mx_ref, jnp.max(v))


def _edge_kernel(edge_attr, we_ae):
    return pl.pallas_call(
        _edge_body,
        grid=(E // _EBLK,),
        in_specs=[pl.BlockSpec((_EBLK, 2), lambda i: (i, 0)), _fs((1, 2))],
        out_specs=[pl.BlockSpec((_EBLK, 1), lambda i: (i, 0)),
                   pl.BlockSpec((1, 1), lambda i: (0, 0))],
        out_shape=[jax.ShapeDtypeStruct((E, 1), jnp.float32),
                   jax.ShapeDtypeStruct((1, 1), jnp.float32)],
    )(edge_attr, we_ae[None, :])


def _final_body(u_ref, den_ref, isum_ref, icnt_ref, gb_ref, wl_ref, wr_ref,
                b4_ref, lw_ref, lb_ref, llw_ref, llb_ref, logit_ref):
    den = jnp.maximum(den_ref[...][:, 0:1], 1e-16)
    s3 = jnp.maximum(u_ref[...] / den + gb_ref[...], 0.0)
    icnt = jnp.maximum(icnt_ref[...][:, 0:1], 1.0)
    mean = isum_ref[...] / icnt
    s4 = jnp.maximum(jnp.dot(mean, wl_ref[...])
                     + jnp.dot(s3, wr_ref[...]) + b4_ref[...], 0.0)
    s5 = jnp.dot(s4, lw_ref[...]) + lb_ref[...]
    logit_ref[...] = jnp.dot(s5, llw_ref[...]) + llb_ref[...]


def _final_kernel(u, den16, in_sum, icnt8, gb, wl, wr, b4, lw, lb, llw, llb):
    return pl.pallas_call(
        _final_body,
        grid=(_NBLK,),
        in_specs=[_rs(64), _rs(16), _rs(64), _rs(8), _fs((1, 64)),
                  _fs((64, 64)), _fs((64, 64)), _fs((1, 64)), _fs((64, 8)),
                  _fs((1, 8)), _fs((8, 1)), _fs((1, 1))],
        out_specs=_rs(1),
        out_shape=jax.ShapeDtypeStruct((NS, 1), jnp.float32),
    )(u, den16, in_sum, icnt8, gb[None, :], wl, wr, b4[None, :], lw,
      lb[None, :], llw, llb[None, :])


def kernel(x_game, x_state, edge_attr, params, edge_index_gg, edge_index_ss, edge_index_hist, edge_index_in):
    p = params
    e2 = lambda v: v
    _Z64 = jnp.zeros((ZCH, 64), jnp.float32)
    _Z16 = jnp.zeros((ZCH, 16), jnp.float32)
    _Z8 = jnp.zeros((ZCH, 8), jnp.float32)
    _ONES8 = jnp.ones((128, 8), jnp.float32)
    fold = lambda c: c[:, 0]

    deg8 = _sum2(_seg_count(e2(edge_index_gg[1]), _Z8, _ONES8))
    dis = _dis_kernel(deg8)
    x16 = _pad16(x_game)
    h1 = _sum2(_seg_norm_16(x16, dis, e2(edge_index_gg[0]),
                            e2(edge_index_gg[1]), _Z16))
    h2 = _sum2(_seg_norm_16(h1, dis, e2(edge_index_gg[0]),
                            e2(edge_index_gg[1]), _Z16))

    s_sum2, cnt8p = _seg_plain_16(_pad16(x_state), e2(edge_index_ss[0]),
                                  e2(edge_index_ss[1]), _Z16, _Z8, _ONES8)
    s_sum, cnt8 = _sum2(s_sum2), _sum2(cnt8p)

    g, hs, asn1, asmax = _game_kernel(x_game, h1, h2, p['tag_W'][0],
                                      p['tag_W'][1], p['tag_W'][2],
                                      p['tag_b'], p['g_Ws'], p['g_as'])
    s2, adn1, admax = _state_kernel(s_sum, cnt8, x_state, p['s2_Wl'],
                                    p['s2_Wr'], p['s2_b'],
                                    p['g_Wd'] @ p['g_ad'])
    aen1, aemax = _edge_kernel(edge_attr, p['g_We'] @ p['g_ae'])

    bnd = asmax[0, 0] + admax[0, 0] + aemax[0, 0]
    bnd = jnp.maximum(bnd, 0.2 * bnd)
    bnd16 = jnp.full((16,), bnd)
    asn, adn, aen = asn1[:, 0], adn1[:, 0], aen1[:, 0]
    u = _seg_gat(hs, asn, adn, e2(aen), bnd16,
                 e2(edge_index_hist[0]), e2(edge_index_hist[1]), _Z64)
    den = _sum2(_seg_wcount(asn, adn, e2(aen), bnd16,
                            e2(edge_index_hist[0]), e2(edge_index_hist[1]),
                            _Z16))

    in_sum, icnt8 = _seg_plain_64(g, e2(edge_index_in[0]),
                                  e2(edge_index_in[1]), _Z64, _Z8, _ONES8)

    logit = _final_kernel(u, den, in_sum, icnt8, p['g_b'], p['s4_Wl'],
                          p['s4_Wr'], p['s4_b'], p['lin_W'], p['lin_b'],
                          p['ll_W'], p['ll_b'])
    state_out = _softmax_over_all(logit[:, 0])[:, None]
    return (state_out, x_game)
